# baseline (speedup 1.0000x reference)
import functools

import jax
import jax.numpy as jnp
from jax import lax
from jax.experimental import pallas as pl
from jax.experimental.pallas import tpu as pltpu
from jax.experimental.pallas import tpu_sc as plsc

_NC, _NS = 2, 16
_NW = _NC * _NS
_LANES = 16
_K = 16


def kernel(x, pos_table):
    B, L, D = x.shape
    xf = x.reshape(B * L * D)
    tf = pos_table.reshape(-1)[: L * D]

    pw = L // _NW
    steps = pw // _K
    chunk = _K * D

    mesh = plsc.VectorSubcoreMesh(core_axis_name="c", subcore_axis_name="s")

    @functools.partial(
        pl.kernel,
        out_type=jax.ShapeDtypeStruct((B * L * D,), jnp.float32),
        mesh=mesh,
        scratch_types=[
            pltpu.VMEM((chunk,), jnp.float32),
            pltpu.VMEM((chunk,), jnp.float32),
        ],
    )
    def sc_add(x_hbm, tab_hbm, out_hbm, tbuf, obuf):
        wid = lax.axis_index("s") * _NC + lax.axis_index("c")
        pos0 = wid * pw
        for j in range(steps):
            prow = pos0 + j * _K
            pltpu.sync_copy(tab_hbm.at[pl.ds(prow * D, chunk)], tbuf)
            for b in range(B):
                base = (b * L + prow) * D
                pltpu.sync_copy(x_hbm.at[pl.ds(base, chunk)], obuf)

                def body(i, _):
                    sl = pl.ds(i * _LANES, _LANES)
                    plsc.addupdate(obuf.at[sl], tbuf[sl])
                    return _

                lax.fori_loop(0, chunk // _LANES, body, 0)
                pltpu.sync_copy(obuf, out_hbm.at[pl.ds(base, chunk)])

    return sc_add(xf, tf).reshape(B, L, D)


# speedup vs baseline: 1.5040x; 1.5040x over previous
import functools

import jax
import jax.numpy as jnp
from jax import lax
from jax.experimental import pallas as pl
from jax.experimental.pallas import tpu as pltpu
from jax.experimental.pallas import tpu_sc as plsc

_NC, _NS = 2, 16
_NW = _NC * _NS
_LANES = 16
_K = 8
_UNROLL = 4


def kernel(x, pos_table):
    B, L, D = x.shape
    xf = x.reshape(B * L * D)
    tf = pos_table.reshape(-1)[: L * D]

    pw = L // _NW
    steps = pw // _K
    chunk = _K * D
    nvec = chunk // _LANES

    mesh = plsc.VectorSubcoreMesh(core_axis_name="c", subcore_axis_name="s")

    @functools.partial(
        pl.kernel,
        out_type=jax.ShapeDtypeStruct((B * L * D,), jnp.float32),
        mesh=mesh,
        scratch_types=[
            pltpu.VMEM((2, chunk), jnp.float32),
            pltpu.VMEM((2, B * chunk), jnp.float32),
            pltpu.SemaphoreType.DMA,
            pltpu.SemaphoreType.DMA,
            pltpu.SemaphoreType.DMA,
        ],
    )
    def sc_add(x_hbm, tab_hbm, out_hbm, tbuf, obuf, sem_t, sem_x, sem_o):
        wid = lax.axis_index("s") * _NC + lax.axis_index("c")
        pos0 = wid * pw

        def issue_ins(j, p):
            prow = pos0 + j * _K
            pltpu.async_copy(tab_hbm.at[pl.ds(prow * D, chunk)],
                             tbuf.at[p], sem_t)
            for b in range(B):
                pltpu.async_copy(x_hbm.at[pl.ds((b * L + prow) * D, chunk)],
                                 obuf.at[p, pl.ds(b * chunk, chunk)], sem_x)

        def wait_ins(p):
            pltpu.make_async_copy(tab_hbm.at[pl.ds(0, chunk)],
                                  tbuf.at[p], sem_t).wait()
            pltpu.make_async_copy(x_hbm.at[pl.ds(0, B * chunk)],
                                  obuf.at[p], sem_x).wait()

        def issue_outs(j, p):
            prow = pos0 + j * _K
            for b in range(B):
                pltpu.async_copy(obuf.at[p, pl.ds(b * chunk, chunk)],
                                 out_hbm.at[pl.ds((b * L + prow) * D, chunk)],
                                 sem_o)

        def wait_outs(p):
            pltpu.make_async_copy(x_hbm.at[pl.ds(0, B * chunk)],
                                  obuf.at[p], sem_o).wait()

        def compute(p):
            def body(i, _):
                base = i * (_LANES * _UNROLL)
                for u in range(_UNROLL):
                    off = base + u * _LANES
                    v = tbuf[p, pl.ds(off, _LANES)]
                    for b in range(B):
                        plsc.addupdate(
                            obuf.at[p, pl.ds(b * chunk + off, _LANES)], v)
                return _

            lax.fori_loop(0, nvec // _UNROLL, body, 0)

        def sub(j, p):
            wait_outs(1 - p)
            issue_ins(j + 1, 1 - p)
            wait_ins(p)
            compute(p)
            issue_outs(j, p)

        issue_ins(0, 0)
        wait_ins(0)
        compute(0)
        issue_outs(0, 0)
        issue_ins(1, 1)

        def pair(j2, _):
            j = 1 + j2 * 2
            sub(j, 1)
            sub(j + 1, 0)
            return _

        lax.fori_loop(0, (steps - 2) // 2, pair, 0)

        wait_outs(0)
        wait_ins(1)
        compute(1)
        issue_outs(steps - 1, 1)
        wait_outs(1)

    return sc_add(xf, tf).reshape(B, L, D)


# speedup vs baseline: 4.6165x; 3.0696x over previous
import functools

import jax
import jax.numpy as jnp
from jax import lax
from jax.experimental import pallas as pl
from jax.experimental.pallas import tpu as pltpu
from jax.experimental.pallas import tpu_sc as plsc

_NC, _NS = 2, 16
_NW = _NC * _NS
_LANES = 16
_K = 8


def kernel(x, pos_table):
    B, L, D = x.shape
    tab = pos_table[:L]

    pw = L // _NW
    steps = pw // _K
    nvec = D // _LANES

    mesh = plsc.VectorSubcoreMesh(core_axis_name="c", subcore_axis_name="s")

    @functools.partial(
        pl.kernel,
        out_type=jax.ShapeDtypeStruct((B, L, D), jnp.float32),
        mesh=mesh,
        scratch_types=[
            pltpu.VMEM((2, _K, D), jnp.float32),
            pltpu.VMEM((2, B, _K, D), jnp.float32),
            pltpu.SemaphoreType.DMA,
            pltpu.SemaphoreType.DMA,
            pltpu.SemaphoreType.DMA,
        ],
    )
    def sc_add(x_hbm, tab_hbm, out_hbm, tbuf, obuf, sem_t, sem_x, sem_o):
        wid = lax.axis_index("s") * _NC + lax.axis_index("c")
        pos0 = wid * pw

        def issue_ins(j, p):
            prow = pos0 + j * _K
            pltpu.async_copy(tab_hbm.at[pl.ds(prow, _K), :],
                             tbuf.at[p], sem_t)
            for b in range(B):
                pltpu.async_copy(x_hbm.at[b, pl.ds(prow, _K), :],
                                 obuf.at[p, b], sem_x)

        def wait_ins(p):
            pltpu.make_async_copy(tab_hbm.at[pl.ds(0, _K), :],
                                  tbuf.at[p], sem_t).wait()
            pltpu.make_async_copy(x_hbm.at[:, pl.ds(0, _K), :],
                                  obuf.at[p], sem_x).wait()

        def issue_outs(j, p):
            prow = pos0 + j * _K
            for b in range(B):
                pltpu.async_copy(obuf.at[p, b],
                                 out_hbm.at[b, pl.ds(prow, _K), :], sem_o)

        def wait_outs(p):
            pltpu.make_async_copy(x_hbm.at[:, pl.ds(0, _K), :],
                                  obuf.at[p], sem_o).wait()

        def compute(p):
            def body(i, _):
                off = i * _LANES
                for r in range(_K):
                    v = tbuf[p, r, pl.ds(off, _LANES)]
                    for b in range(B):
                        plsc.addupdate(obuf.at[p, b, r, pl.ds(off, _LANES)], v)
                return _

            lax.fori_loop(0, nvec, body, 0)

        def sub(j, p):
            wait_outs(1 - p)
            issue_ins(j + 1, 1 - p)
            wait_ins(p)
            compute(p)
            issue_outs(j, p)

        issue_ins(0, 0)
        wait_ins(0)
        compute(0)
        issue_outs(0, 0)
        issue_ins(1, 1)

        def pair(j2, _):
            j = 1 + j2 * 2
            sub(j, 1)
            sub(j + 1, 0)
            return _

        lax.fori_loop(0, (steps - 2) // 2, pair, 0)

        wait_outs(0)
        wait_ins(1)
        compute(1)
        issue_outs(steps - 1, 1)
        wait_outs(1)

    return sc_add(x, tab)
